# baseline (speedup 1.0000x reference)
import jax
import jax.numpy as jnp
from jax.experimental import pallas as pl
from jax.experimental.pallas import tpu as pltpu

_B, _N, _CF, _T, _H, _W = 4, 8, 128, 16, 32, 32
_CPOS = 64


def _tc_body(bbox_sref, frames_ref, wposT_ref, out_ref):
    i = pl.program_id(0)
    t = pl.program_id(1)
    x0 = bbox_sref[i, t, 0]
    y0 = bbox_sref[i, t, 1]
    x2 = bbox_sref[i, t, 2]
    y2 = bbox_sref[i, t, 3]
    hw = jax.lax.broadcasted_iota(jnp.int32, (1, _H * _W), 1)
    h = hw // _W
    w = hw % _W
    mask = ((h >= x0) & (h < x2) & (w >= y0) & (w < y2)).astype(jnp.float32)
    count = jnp.sum(mask)
    frames = frames_ref[0]
    s = jax.lax.dot_general(
        mask, frames,
        dimension_numbers=(((1,), (1,)), ((), ())),
        preferred_element_type=jnp.float32,
    )
    safe = jnp.maximum(count, 1.0)
    degenerate = (x0 >= x2) & (y0 >= y2)
    valid = (count > 0) & (~degenerate)
    pix = jnp.where(valid, s / safe, 0.0)
    wposT = wposT_ref[...]
    pos = (
        x0.astype(jnp.float32) * wposT[0:1]
        + y0.astype(jnp.float32) * wposT[1:2]
        + x2.astype(jnp.float32) * wposT[2:3]
        + y2.astype(jnp.float32) * wposT[3:4]
    )
    out_ref[0, 0, :, : _CF] = pix
    out_ref[0, 0, :, _CF:] = pos


def kernel(frames, bbox, W_pos):
    bn = _B * _N
    frames2 = frames.reshape(bn, _CF, _T * _H * _W)
    bbox2 = bbox.reshape(bn, _T, 4)
    grid_spec = pltpu.PrefetchScalarGridSpec(
        num_scalar_prefetch=1,
        grid=(bn, _T),
        in_specs=[
            pl.BlockSpec((1, _CF, _H * _W), lambda i, t, sp: (i, 0, t)),
            pl.BlockSpec((4, _CPOS), lambda i, t, sp: (0, 0)),
        ],
        out_specs=pl.BlockSpec((1, 1, 1, _CF + _CPOS), lambda i, t, sp: (i, t, 0, 0)),
    )
    out = pl.pallas_call(
        _tc_body,
        grid_spec=grid_spec,
        out_shape=jax.ShapeDtypeStruct((bn, _T, 1, _CF + _CPOS), jnp.float32),
    )(bbox2, frames2, W_pos.T)
    return out.reshape(_B, _N, _T, _CF + _CPOS)
